# Initial kernel scaffold; baseline (speedup 1.0000x reference)
import jax
import jax.numpy as jnp
from jax.experimental import pallas as pl


def kernel(queries, memory_bank):
    raise NotImplementedError("write your pallas kernel here")


# speedup vs baseline: 5.2335x; 5.2335x over previous
import jax
import jax.numpy as jnp
from jax.experimental import pallas as pl
from jax.experimental.pallas import tpu as pltpu

_Q = 1024
_K = 50000
_D = 768
_KB = 2000
_NBLK = _K // _KB


def _knn_block(q_ref, mb_ref, out_ref, acc_ref):
    i = pl.program_id(0)
    mb = mb_ref[...]
    m_sq = jnp.sum(mb * mb, axis=1, keepdims=True)
    q16 = q_ref[...].astype(jnp.bfloat16)
    mb16 = mb.astype(jnp.bfloat16)
    dots = jax.lax.dot_general(
        q16, mb16, (((1,), (1,)), ((), ())),
        preferred_element_type=jnp.float32)
    scores = dots - 0.5 * m_sq.reshape(1, _KB)
    blk_max = jnp.max(scores, axis=1, keepdims=True)

    @pl.when(i == 0)
    def _init():
        acc_ref[...] = blk_max

    @pl.when(i > 0)
    def _accum():
        acc_ref[...] = jnp.maximum(acc_ref[...], blk_max)

    @pl.when(i == _NBLK - 1)
    def _finish():
        q = q_ref[...]
        q_sq = jnp.sum(q * q, axis=1, keepdims=True)
        d2 = q_sq - 2.0 * acc_ref[...]
        out_ref[...] = jnp.sqrt(jnp.maximum(d2, 1e-12))


def kernel(queries, memory_bank):
    out = pl.pallas_call(
        _knn_block,
        grid=(_NBLK,),
        in_specs=[
            pl.BlockSpec((_Q, _D), lambda i: (0, 0)),
            pl.BlockSpec((_KB, _D), lambda i: (i, 0)),
        ],
        out_specs=pl.BlockSpec((_Q, 1), lambda i: (0, 0)),
        out_shape=jax.ShapeDtypeStruct((_Q, 1), jnp.float32),
        scratch_shapes=[pltpu.VMEM((_Q, 1), jnp.float32)],
        compiler_params=pltpu.CompilerParams(
            dimension_semantics=("arbitrary",)),
    )(queries, memory_bank)
    return out[:, 0]
